# baseline (speedup 1.0000x reference)
import jax
import jax.numpy as jnp
from jax.experimental import pallas as pl
from jax.experimental.pallas import tpu as pltpu


def _round_up(n, m):
    return ((n + m - 1) // m) * m


def _linear_kernel(x_ref, w_ref, b_ref, o_ref):
    k = pl.program_id(1)
    TBH = o_ref.shape[0]
    xs = x_ref[pl.ds(k * TBH, TBH), :]
    acc = jax.lax.dot_general(
        xs, w_ref[...], (((1,), (1,)), ((), ())),
        preferred_element_type=jnp.float32)
    o_ref[...] = acc + b_ref[...]


def kernel(x, weight, bias):
    B, D = x.shape
    C, D2 = weight.shape
    assert D == D2 and bias.shape == (C,)

    CPAD = _round_up(C, 128)

    TB = _round_up(B, 8)
    while TB > 8 and (2 * TB * (D + CPAD) * 4 + 2 * CPAD * D * 4
                      > 48 * 1024 * 1024 or TB * 2 > _round_up(B, 8)):
        TB = _round_up(TB // 2, 8)
    B_pad = _round_up(B, TB)

    x = x.astype(jnp.float32)
    x_p = x if B_pad == B else jnp.pad(x, ((0, B_pad - B), (0, 0)))
    w_p = weight.astype(jnp.float32)
    if CPAD != C:
        w_p = jnp.pad(w_p, ((0, CPAD - C), (0, 0)))
    b_row = jnp.pad(bias.astype(jnp.float32), (0, CPAD - C)).reshape(1, CPAD)

    cost = pl.CostEstimate(
        flops=2 * B * D * C,
        transcendentals=0,
        bytes_accessed=int(B_pad * D * 4 + D * CPAD * 4
                           + CPAD * 4 + B_pad * CPAD * 4),
    )

    out_padded = pl.pallas_call(
        _linear_kernel,
        out_shape=jax.ShapeDtypeStruct((B_pad, CPAD), jnp.float32),
        grid_spec=pltpu.PrefetchScalarGridSpec(
            num_scalar_prefetch=0,
            grid=(B_pad // TB, 2),
            in_specs=[
                pl.BlockSpec((TB, D), lambda i, k: (i, 0)),
                pl.BlockSpec((CPAD, D), lambda i, k: (0, 0)),
                pl.BlockSpec((1, CPAD), lambda i, k: (0, 0)),
            ],
            out_specs=pl.BlockSpec((TB // 2, CPAD), lambda i, k: (2 * i + k, 0)),
        ),
        compiler_params=pltpu.CompilerParams(
            dimension_semantics=("parallel", "arbitrary"),
            vmem_limit_bytes=56 * 1024 * 1024),
        cost_estimate=cost,
    )(x_p, w_p, b_row)

    return out_padded[:B, :C]


# speedup vs baseline: 1.1389x; 1.1389x over previous
import jax
import jax.numpy as jnp
from jax.experimental import pallas as pl
from jax.experimental.pallas import tpu as pltpu


def _round_up(n, m):
    return ((n + m - 1) // m) * m


def _linear_kernel(x_ref, w_ref, b_ref, o_ref):
    acc = jax.lax.dot_general(
        x_ref[...], w_ref[...], (((1,), (1,)), ((), ())),
        preferred_element_type=jnp.float32)
    o_ref[...] = acc + b_ref[...]


def kernel(x, weight, bias):
    B, D = x.shape
    C, D2 = weight.shape
    assert D == D2 and bias.shape == (C,)

    CPAD = _round_up(C, 128)

    TB = _round_up(B, 8)
    while TB > 8 and (2 * TB * (D + CPAD) * 4 + 2 * CPAD * D * 4
                      > 48 * 1024 * 1024 or TB * 2 > _round_up(B, 8)):
        TB = _round_up(TB // 2, 8)
    B_pad = _round_up(B, TB)

    x = x.astype(jnp.float32)
    x_p = x if B_pad == B else jnp.pad(x, ((0, B_pad - B), (0, 0)))
    w_p = weight.astype(jnp.float32)
    if CPAD != C:
        w_p = jnp.pad(w_p, ((0, CPAD - C), (0, 0)))
    b_row = jnp.pad(bias.astype(jnp.float32), (0, CPAD - C)).reshape(1, CPAD)

    cost = pl.CostEstimate(
        flops=2 * B * D * C,
        transcendentals=0,
        bytes_accessed=int(B_pad * D * 4 + D * CPAD * 4
                           + CPAD * 4 + B_pad * CPAD * 4),
    )

    out_padded = pl.pallas_call(
        _linear_kernel,
        out_shape=jax.ShapeDtypeStruct((B_pad, CPAD), jnp.float32),
        grid_spec=pltpu.PrefetchScalarGridSpec(
            num_scalar_prefetch=0,
            grid=(B_pad // TB,),
            in_specs=[
                pl.BlockSpec((TB, D), lambda i: (i, 0)),
                pl.BlockSpec((CPAD, D), lambda i: (0, 0)),
                pl.BlockSpec((1, CPAD), lambda i: (0, 0)),
            ],
            out_specs=pl.BlockSpec((TB, CPAD), lambda i: (i, 0)),
        ),
        compiler_params=pltpu.CompilerParams(
            dimension_semantics=("parallel",),
            vmem_limit_bytes=56 * 1024 * 1024),
        cost_estimate=cost,
    )(x_p, w_p, b_row)

    return out_padded[:B, :C]
